# baseline (speedup 1.0000x reference)
import jax
import jax.numpy as jnp
from jax.experimental import pallas as pl
from jax.experimental.pallas import tpu as pltpu

_N = 10000
_IN = 128
_OUT = 128
_T = 4
_R = 1000


def _body(t_ref, x_ref, wc_ref, b_ref, o_ref):
    xv = x_ref[...]
    tv = t_ref[...]
    y = jnp.dot(xv, wc_ref[...], preferred_element_type=jnp.float32)
    out = y[:, 3 * _OUT:]
    bias = b_ref[3]
    for t in (2, 1, 0):
        sel = tv == t
        out = jnp.where(sel, y[:, t * _OUT:(t + 1) * _OUT], out)
        bias = jnp.where(sel, b_ref[t], bias)
    o_ref[...] = out + bias


def kernel(x, edge_index, node_types, W, b):
    del edge_index
    nt = node_types.reshape(_N, 1)
    wc = W.transpose(1, 0, 2).reshape(_IN, _T * _OUT)
    return pl.pallas_call(
        _body,
        grid=(_N // _R,),
        in_specs=[
            pl.BlockSpec((_R, 1), lambda i: (i, 0)),
            pl.BlockSpec((_R, _IN), lambda i: (i, 0)),
            pl.BlockSpec((_IN, _T * _OUT), lambda i: (0, 0)),
            pl.BlockSpec((_T, _OUT), lambda i: (0, 0)),
        ],
        out_specs=pl.BlockSpec((_R, _OUT), lambda i: (i, 0)),
        out_shape=jax.ShapeDtypeStruct((_N, _OUT), jnp.float32),
        compiler_params=pltpu.CompilerParams(
            dimension_semantics=("parallel",),
        ),
    )(nt, x, wc, b)


# speedup vs baseline: 1.0589x; 1.0589x over previous
import jax
import jax.numpy as jnp
from jax.experimental import pallas as pl
from jax.experimental.pallas import tpu as pltpu

_N = 10000
_IN = 128
_OUT = 128
_T = 4
_C = 10
_R = _N // _C


def _body(t_hbm, x_hbm, wc_hbm, b_hbm, o_hbm,
          t_v, x_v, wc_v, b_v, o_v,
          in_sems, aux_sem, out_sems):
    aux_copies = [
        pltpu.make_async_copy(wc_hbm, wc_v, aux_sem.at[0]),
        pltpu.make_async_copy(b_hbm, b_v, aux_sem.at[1]),
        pltpu.make_async_copy(t_hbm, t_v, aux_sem.at[2]),
    ]
    for c in aux_copies:
        c.start()
    in_copies = []
    for i in range(_C):
        sl = pl.ds(i * _R, _R)
        c = pltpu.make_async_copy(x_hbm.at[sl, :], x_v.at[sl, :], in_sems.at[i])
        c.start()
        in_copies.append(c)
    for c in aux_copies:
        c.wait()

    out_copies = []
    for i in range(_C):
        sl = pl.ds(i * _R, _R)
        in_copies[i].wait()
        xv = x_v[sl, :]
        tv = t_v[sl, :]
        y = jnp.dot(xv, wc_v[...], preferred_element_type=jnp.float32)
        out = y[:, 3 * _OUT:]
        bias = b_v[3]
        for t in (2, 1, 0):
            sel = tv == t
            out = jnp.where(sel, y[:, t * _OUT:(t + 1) * _OUT], out)
            bias = jnp.where(sel, b_v[t], bias)
        o_v[sl, :] = out + bias
        c = pltpu.make_async_copy(o_v.at[sl, :], o_hbm.at[sl, :], out_sems.at[i])
        c.start()
        out_copies.append(c)
    for c in out_copies:
        c.wait()


def kernel(x, edge_index, node_types, W, b):
    del edge_index
    nt = node_types.reshape(_N, 1)
    wc = W.transpose(1, 0, 2).reshape(_IN, _T * _OUT)
    return pl.pallas_call(
        _body,
        in_specs=[
            pl.BlockSpec(memory_space=pl.ANY),
            pl.BlockSpec(memory_space=pl.ANY),
            pl.BlockSpec(memory_space=pl.ANY),
            pl.BlockSpec(memory_space=pl.ANY),
        ],
        out_specs=pl.BlockSpec(memory_space=pl.ANY),
        out_shape=jax.ShapeDtypeStruct((_N, _OUT), jnp.float32),
        scratch_shapes=[
            pltpu.VMEM((_N, 1), jnp.int32),
            pltpu.VMEM((_N, _IN), jnp.float32),
            pltpu.VMEM((_IN, _T * _OUT), jnp.float32),
            pltpu.VMEM((_T, _OUT), jnp.float32),
            pltpu.VMEM((_N, _OUT), jnp.float32),
            pltpu.SemaphoreType.DMA((_C,)),
            pltpu.SemaphoreType.DMA((3,)),
            pltpu.SemaphoreType.DMA((_C,)),
        ],
    )(nt, x, wc, b)


# speedup vs baseline: 1.1062x; 1.0447x over previous
import jax
import jax.numpy as jnp
from jax.experimental import pallas as pl
from jax.experimental.pallas import tpu as pltpu

_N = 10000
_IN = 128
_OUT = 128
_T = 4
_C = 10
_R = _N // _C


def _body(t_hbm, x_hbm, w_hbm, b_hbm, o_hbm,
          t_v, x_v, wc_v, b_v, o_v,
          in_sems, aux_sem, out_sems):
    aux_copies = [
        pltpu.make_async_copy(b_hbm, b_v, aux_sem.at[0]),
        pltpu.make_async_copy(t_hbm, t_v, aux_sem.at[1]),
    ]
    for t in range(_T):
        aux_copies.append(pltpu.make_async_copy(
            w_hbm.at[t], wc_v.at[:, t * _OUT:(t + 1) * _OUT], aux_sem.at[2 + t]))
    for c in aux_copies:
        c.start()
    in_copies = []
    for i in range(_C):
        sl = pl.ds(i * _R, _R)
        c = pltpu.make_async_copy(x_hbm.at[sl, :], x_v.at[sl, :], in_sems.at[i])
        c.start()
        in_copies.append(c)
    for c in aux_copies:
        c.wait()

    out_copies = []
    for i in range(_C):
        sl = pl.ds(i * _R, _R)
        in_copies[i].wait()
        xv = x_v[sl, :]
        tv = t_v[sl, :]
        y = jnp.dot(xv, wc_v[...], preferred_element_type=jnp.float32)
        out = y[:, 3 * _OUT:]
        bias = b_v[3]
        for t in (2, 1, 0):
            sel = tv == t
            out = jnp.where(sel, y[:, t * _OUT:(t + 1) * _OUT], out)
            bias = jnp.where(sel, b_v[t], bias)
        o_v[sl, :] = out + bias
        c = pltpu.make_async_copy(o_v.at[sl, :], o_hbm.at[sl, :], out_sems.at[i])
        c.start()
        out_copies.append(c)
    for c in out_copies:
        c.wait()


def kernel(x, edge_index, node_types, W, b):
    del edge_index
    nt = node_types.reshape(_N, 1)
    return pl.pallas_call(
        _body,
        in_specs=[
            pl.BlockSpec(memory_space=pl.ANY),
            pl.BlockSpec(memory_space=pl.ANY),
            pl.BlockSpec(memory_space=pl.ANY),
            pl.BlockSpec(memory_space=pl.ANY),
        ],
        out_specs=pl.BlockSpec(memory_space=pl.ANY),
        out_shape=jax.ShapeDtypeStruct((_N, _OUT), jnp.float32),
        scratch_shapes=[
            pltpu.VMEM((_N, 1), jnp.int32),
            pltpu.VMEM((_N, _IN), jnp.float32),
            pltpu.VMEM((_IN, _T * _OUT), jnp.float32),
            pltpu.VMEM((_T, _OUT), jnp.float32),
            pltpu.VMEM((_N, _OUT), jnp.float32),
            pltpu.SemaphoreType.DMA((_C,)),
            pltpu.SemaphoreType.DMA((2 + _T,)),
            pltpu.SemaphoreType.DMA((_C,)),
        ],
    )(nt, x, W, b)


# speedup vs baseline: 2.3897x; 2.1602x over previous
import jax
import jax.numpy as jnp
from jax.experimental import pallas as pl
from jax.experimental.pallas import tpu as pltpu

_N = 10000
_IN = 128
_OUT = 128
_T = 4
_C = 10
_R = _N // _C


def _body(t_hbm, x_hbm, w_hbm, b_hbm, o_hbm,
          t_v, x_v, wc_v, b_v, o_v,
          in_sems, aux_sem, out_sems):
    aux_copies = [
        pltpu.make_async_copy(b_hbm, b_v, aux_sem.at[0]),
        pltpu.make_async_copy(t_hbm, t_v, aux_sem.at[1]),
    ]
    for t in range(_T):
        aux_copies.append(pltpu.make_async_copy(
            w_hbm.at[t], wc_v.at[:, t * _OUT:(t + 1) * _OUT], aux_sem.at[2 + t]))
    for c in aux_copies:
        c.start()
    in_copies = []
    for i in range(_C):
        sl = pl.ds(i * _R, _R)
        c = pltpu.make_async_copy(x_hbm.at[sl, :], x_v.at[sl, :], in_sems.at[i])
        c.start()
        in_copies.append(c)
    for c in aux_copies:
        c.wait()

    out_copies = []
    for i in range(_C):
        sl = pl.ds(i * _R, _R)
        in_copies[i].wait()
        xv = x_v[sl, :]
        tv = t_v[sl].reshape(_R, 1)
        y = jnp.dot(xv, wc_v[...], preferred_element_type=jnp.float32)
        out = y[:, 3 * _OUT:]
        bias = b_v[3]
        for t in (2, 1, 0):
            sel = tv == t
            out = jnp.where(sel, y[:, t * _OUT:(t + 1) * _OUT], out)
            bias = jnp.where(sel, b_v[t], bias)
        o_v[sl, :] = out + bias
        c = pltpu.make_async_copy(o_v.at[sl, :], o_hbm.at[sl, :], out_sems.at[i])
        c.start()
        out_copies.append(c)
    for c in out_copies:
        c.wait()


def kernel(x, edge_index, node_types, W, b):
    del edge_index
    return pl.pallas_call(
        _body,
        in_specs=[
            pl.BlockSpec(memory_space=pl.ANY),
            pl.BlockSpec(memory_space=pl.ANY),
            pl.BlockSpec(memory_space=pl.ANY),
            pl.BlockSpec(memory_space=pl.ANY),
        ],
        out_specs=pl.BlockSpec(memory_space=pl.ANY),
        out_shape=jax.ShapeDtypeStruct((_N, _OUT), jnp.float32),
        scratch_shapes=[
            pltpu.VMEM((_N,), jnp.int32),
            pltpu.VMEM((_N, _IN), jnp.float32),
            pltpu.VMEM((_IN, _T * _OUT), jnp.float32),
            pltpu.VMEM((_T, _OUT), jnp.float32),
            pltpu.VMEM((_N, _OUT), jnp.float32),
            pltpu.SemaphoreType.DMA((_C,)),
            pltpu.SemaphoreType.DMA((2 + _T,)),
            pltpu.SemaphoreType.DMA((_C,)),
        ],
    )(node_types, x, W, b)
